# baseline (speedup 1.0000x reference)
import functools

import jax
import jax.numpy as jnp
from jax import lax
from jax.experimental import pallas as pl
from jax.experimental.pallas import tpu as pltpu
from jax.experimental.pallas import tpu_sc as plsc

_N = 1_000_000
_D = 64
_Q = 64
_NP = _N // 2
_BLKP = 5_000
_EPS = 1e-12


def _norm_rows(x):
    return x / jnp.maximum(
        jnp.sqrt(jnp.sum(x * x, axis=1, keepdims=True)), _EPS)


def _topk_body(q_ref, mp_ref, conf_ref, idx_ref):
    i = pl.program_id(0)

    @pl.when(i == 0)
    def _init():
        conf_ref[...] = jnp.full((1, _Q), -jnp.inf, jnp.float32)
        idx_ref[...] = jnp.zeros((1, _Q), jnp.int32)

    qn = _norm_rows(q_ref[...])
    mp = mp_ref[...]
    mn_e = _norm_rows(mp[:, :_D])
    mn_o = _norm_rows(mp[:, _D:])
    sims_e = lax.dot_general(qn, mn_e, (((1,), (1,)), ((), ())),
                             preferred_element_type=jnp.float32)
    sims_o = lax.dot_general(qn, mn_o, (((1,), (1,)), ((), ())),
                             preferred_element_type=jnp.float32)

    col = lax.broadcasted_iota(jnp.int32, sims_e.shape, 1)
    vmax_e = jnp.max(sims_e, axis=1)
    arg_e = jnp.min(jnp.where(sims_e == vmax_e[:, None], col, _BLKP), axis=1)
    vmax_o = jnp.max(sims_o, axis=1)
    arg_o = jnp.min(jnp.where(sims_o == vmax_o[:, None], col, _BLKP), axis=1)

    base = i * _BLKP
    ge = 2 * (base + arg_e)
    go = 2 * (base + arg_o) + 1
    take_o = (vmax_o > vmax_e) | ((vmax_o == vmax_e) & (go < ge))
    v = jnp.where(take_o, vmax_o, vmax_e)
    g = jnp.where(take_o, go, ge)

    run_v = conf_ref[0, :]
    upd = v > run_v
    conf_ref[0, :] = jnp.where(upd, v, run_v)
    idx_ref[0, :] = jnp.where(upd, g, idx_ref[0, :])


_topk_call = pl.pallas_call(
    _topk_body,
    grid=(_NP // _BLKP,),
    in_specs=[
        pl.BlockSpec((_Q, _D), lambda i: (0, 0)),
        pl.BlockSpec((_BLKP, 2 * _D), lambda i: (i, 0)),
    ],
    out_specs=[
        pl.BlockSpec((1, _Q), lambda i: (0, 0)),
        pl.BlockSpec((1, _Q), lambda i: (0, 0)),
    ],
    out_shape=[
        jax.ShapeDtypeStruct((1, _Q), jnp.float32),
        jax.ShapeDtypeStruct((1, _Q), jnp.int32),
    ],
)


def _sc_gather_body(values_hbm, idx_hbm, out_hbm, idx_v, rows_v, sem):
    wid = lax.axis_index("s") * 2 + lax.axis_index("c")

    @pl.when(wid == 0)
    def _():
        pltpu.sync_copy(idx_hbm, idx_v)
        pltpu.async_copy(values_hbm.at[idx_v], rows_v, sem).wait()
        pltpu.sync_copy(rows_v, out_hbm)


_sc_gather = functools.partial(
    pl.kernel,
    out_type=jax.ShapeDtypeStruct((_Q,), jnp.int32),
    mesh=plsc.VectorSubcoreMesh(core_axis_name="c", subcore_axis_name="s"),
    scratch_types=[
        pltpu.VMEM((_Q,), jnp.int32),
        pltpu.VMEM((_Q,), jnp.int32),
        pltpu.SemaphoreType.DMA,
    ],
)(_sc_gather_body)


def kernel(query_features, memory_keys, memory_values):
    mp = memory_keys.reshape(_NP, 2 * _D)
    conf2, idx2 = _topk_call(query_features, mp)
    confidence = conf2[0]
    indices = idx2[0]
    retrieved = _sc_gather(memory_values, indices)
    return retrieved, confidence


# speedup vs baseline: 1.4941x; 1.4941x over previous
import functools

import jax
import jax.numpy as jnp
from jax import lax
from jax.experimental import pallas as pl
from jax.experimental.pallas import tpu as pltpu
from jax.experimental.pallas import tpu_sc as plsc

_N = 1_000_000
_D = 64
_Q = 64
_NP = _N // 2
_BLKP = 5_000
_EPS = 1e-12


def _norm_rows(x):
    return x / jnp.maximum(
        jnp.sqrt(jnp.sum(x * x, axis=1, keepdims=True)), _EPS)


_BLK = 10_000


def _topk_body(q_ref, m_hbm, conf_ref, idx_ref, mbuf, sems):
    i = pl.program_id(0)
    n = pl.num_programs(0)
    slot = lax.rem(i, 2)

    @pl.when(i == 0)
    def _init():
        conf_ref[...] = jnp.full((1, _Q), -jnp.inf, jnp.float32)
        idx_ref[...] = jnp.zeros((1, _Q), jnp.int32)
        pltpu.make_async_copy(
            m_hbm.at[pl.ds(0, _BLK)], mbuf.at[0], sems.at[0]).start()

    @pl.when(i + 1 < n)
    def _prefetch():
        nxt = lax.rem(i + 1, 2)
        pltpu.make_async_copy(
            m_hbm.at[pl.ds((i + 1) * _BLK, _BLK)], mbuf.at[nxt],
            sems.at[nxt]).start()

    pltpu.make_async_copy(
        m_hbm.at[pl.ds(i * _BLK, _BLK)], mbuf.at[slot], sems.at[slot]).wait()

    qn = _norm_rows(q_ref[...])
    m = mbuf[slot]
    mn = _norm_rows(m)
    sims = lax.dot_general(qn, mn, (((1,), (1,)), ((), ())),
                           preferred_element_type=jnp.float32)

    local_max = jnp.max(sims, axis=1)
    col = lax.broadcasted_iota(jnp.int32, sims.shape, 1)
    local_arg = jnp.min(
        jnp.where(sims == local_max[:, None], col, _BLK), axis=1)

    run_v = conf_ref[0, :]
    upd = local_max > run_v
    conf_ref[0, :] = jnp.where(upd, local_max, run_v)
    idx_ref[0, :] = jnp.where(upd, i * _BLK + local_arg, idx_ref[0, :])


_topk_call = pl.pallas_call(
    _topk_body,
    grid=(_N // _BLK,),
    in_specs=[
        pl.BlockSpec((_Q, _D), lambda i: (0, 0)),
        pl.BlockSpec(memory_space=pl.ANY),
    ],
    out_specs=[
        pl.BlockSpec((1, _Q), lambda i: (0, 0)),
        pl.BlockSpec((1, _Q), lambda i: (0, 0)),
    ],
    out_shape=[
        jax.ShapeDtypeStruct((1, _Q), jnp.float32),
        jax.ShapeDtypeStruct((1, _Q), jnp.int32),
    ],
    scratch_shapes=[
        pltpu.VMEM((2, _BLK, _D), jnp.float32),
        pltpu.SemaphoreType.DMA((2,)),
    ],
)


def _sc_gather_body(values_hbm, idx_hbm, out_hbm, idx_v, rows_v, sem):
    wid = lax.axis_index("s") * 2 + lax.axis_index("c")

    @pl.when(wid == 0)
    def _():
        pltpu.sync_copy(idx_hbm, idx_v)
        pltpu.async_copy(values_hbm.at[idx_v], rows_v, sem).wait()
        pltpu.sync_copy(rows_v, out_hbm)


_sc_gather = functools.partial(
    pl.kernel,
    out_type=jax.ShapeDtypeStruct((_Q,), jnp.int32),
    mesh=plsc.VectorSubcoreMesh(core_axis_name="c", subcore_axis_name="s"),
    scratch_types=[
        pltpu.VMEM((_Q,), jnp.int32),
        pltpu.VMEM((_Q,), jnp.int32),
        pltpu.SemaphoreType.DMA,
    ],
)(_sc_gather_body)


def kernel(query_features, memory_keys, memory_values):
    conf2, idx2 = _topk_call(query_features, memory_keys)
    confidence = conf2[0]
    indices = idx2[0]
    retrieved = _sc_gather(memory_values, indices)
    return retrieved, confidence
